# baseline (speedup 1.0000x reference)
import functools

import jax
import jax.numpy as jnp
from jax import lax
from jax.experimental import pallas as pl
from jax.experimental.pallas import tpu as pltpu
from jax.experimental.pallas import tpu_sc as plsc

_ROWS = 98304
_LANES = 128
_SC_ROWS = 49152
_NW = 32
_ROWS_PER_W = _SC_ROWS // _NW
_CHUNK = 192
_NCHUNK = _ROWS_PER_W // _CHUNK
_NBUF = 4
_TC_BLOCK = 8192
_TC_BLOCK0 = _SC_ROWS // _TC_BLOCK
_TC_GRID = (_ROWS - _SC_ROWS) // _TC_BLOCK


def _make_sc_copy():
    mesh = plsc.VectorSubcoreMesh(core_axis_name="c", subcore_axis_name="s")

    @functools.partial(
        pl.kernel,
        mesh=mesh,
        out_type=jax.ShapeDtypeStruct((_ROWS, _LANES), jnp.float32),
        scratch_types=(
            [pltpu.MemorySpace.VMEM_SHARED(
                (16, _NBUF, _CHUNK, _LANES), jnp.float32)]
            + [pltpu.SemaphoreType.DMA for _ in range(2 * _NBUF)]
        ),
    )
    def sc_copy(x_hbm, o_hbm, *scratch):
        shared = scratch[0]
        rsems = scratch[1:1 + _NBUF]
        wsems = scratch[1 + _NBUF:]
        sid = lax.axis_index("s")
        wid = sid * 2 + lax.axis_index("c")
        base = wid * _ROWS_PER_W

        def rd(g):
            return pltpu.make_async_copy(
                x_hbm.at[pl.ds(base + g * _CHUNK, _CHUNK)],
                shared.at[sid, g % _NBUF], rsems[g % _NBUF])

        def wr(g):
            return pltpu.make_async_copy(
                shared.at[sid, g % _NBUF],
                o_hbm.at[pl.ds(base + g * _CHUNK, _CHUNK)],
                wsems[g % _NBUF])

        rd(0).start()
        rd(1).start()
        for g in range(_NCHUNK):
            rd(g).wait()
            wr(g).start()
            nxt = g + 2
            if nxt < _NCHUNK:
                if nxt >= _NBUF:
                    wr(nxt - _NBUF).wait()
                rd(nxt).start()
        for g in range(_NCHUNK - _NBUF, _NCHUNK):
            wr(g).wait()

    return sc_copy


_sc_copy = _make_sc_copy()


def _tc_body(x_ref, alias_ref, o_ref):
    del alias_ref
    o_ref[...] = x_ref[...]


def kernel(items):
    n = items.shape[0]
    chunks = n // 128
    flat = (jnp.transpose(items, (1, 3, 0, 2))
            .reshape(2, 3, chunks, 128, 2)
            .transpose(0, 1, 2, 4, 3)
            .reshape(_ROWS, _LANES))
    sc_out = _sc_copy(flat)
    out = pl.pallas_call(
        _tc_body,
        grid=(_TC_GRID,),
        in_specs=[
            pl.BlockSpec((_TC_BLOCK, _LANES), lambda i: (_TC_BLOCK0 + i, 0)),
            pl.BlockSpec(memory_space=pl.ANY),
        ],
        out_specs=pl.BlockSpec((_TC_BLOCK, _LANES),
                               lambda i: (_TC_BLOCK0 + i, 0)),
        out_shape=jax.ShapeDtypeStruct((_ROWS, _LANES), jnp.float32),
        input_output_aliases={1: 0},
    )(flat, sc_out)
    return (out.reshape(2, 3, chunks, 2, 128)
            .transpose(0, 1, 2, 4, 3)
            .reshape(2, 3, n, 2)
            .transpose(2, 0, 3, 1))


# speedup vs baseline: 1.0439x; 1.0439x over previous
import functools

import jax
import jax.numpy as jnp
from jax import lax
from jax.experimental import pallas as pl
from jax.experimental.pallas import tpu as pltpu
from jax.experimental.pallas import tpu_sc as plsc

_ROWS = 98304
_LANES = 128
_NW = 32
_ROWS_PER_W = _ROWS // _NW
_CHUNK = 384
_NCHUNK = _ROWS_PER_W // _CHUNK
_NBUF = 2


def _make_sc_copy():
    mesh = plsc.VectorSubcoreMesh(core_axis_name="c", subcore_axis_name="s")

    @functools.partial(
        pl.kernel,
        mesh=mesh,
        out_type=jax.ShapeDtypeStruct((_ROWS, _LANES), jnp.float32),
        scratch_types=(
            [pltpu.MemorySpace.VMEM_SHARED(
                (16, _NBUF, _CHUNK, _LANES), jnp.float32)]
            + [pltpu.SemaphoreType.DMA for _ in range(2 * _NBUF)]
        ),
    )
    def sc_copy(x_hbm, o_hbm, *scratch):
        shared = scratch[0]
        rsems = scratch[1:1 + _NBUF]
        wsems = scratch[1 + _NBUF:]
        sid = lax.axis_index("s")
        wid = sid * 2 + lax.axis_index("c")
        base = wid * _ROWS_PER_W

        def rd(g):
            return pltpu.make_async_copy(
                x_hbm.at[pl.ds(base + g * _CHUNK, _CHUNK)],
                shared.at[sid, g % _NBUF], rsems[g % _NBUF])

        def wr(g):
            return pltpu.make_async_copy(
                shared.at[sid, g % _NBUF],
                o_hbm.at[pl.ds(base + g * _CHUNK, _CHUNK)],
                wsems[g % _NBUF])

        rd(0).start()
        for g in range(_NCHUNK):
            rd(g).wait()
            wr(g).start()
            nxt = g + 1
            if nxt < _NCHUNK:
                if nxt >= _NBUF:
                    wr(nxt - _NBUF).wait()
                rd(nxt).start()
        for g in range(_NCHUNK - _NBUF, _NCHUNK):
            wr(g).wait()

    return sc_copy


_sc_copy = _make_sc_copy()


def kernel(items):
    n = items.shape[0]
    chunks = n // 128
    flat = (jnp.transpose(items, (1, 3, 0, 2))
            .reshape(2, 3, chunks, 128, 2)
            .transpose(0, 1, 2, 4, 3)
            .reshape(_ROWS, _LANES))
    out = _sc_copy(flat)
    return (out.reshape(2, 3, chunks, 2, 128)
            .transpose(0, 1, 2, 4, 3)
            .reshape(2, 3, n, 2)
            .transpose(2, 0, 3, 1))


# speedup vs baseline: 1.0508x; 1.0067x over previous
import functools

import jax
import jax.numpy as jnp
from jax import lax
from jax.experimental import pallas as pl
from jax.experimental.pallas import tpu as pltpu
from jax.experimental.pallas import tpu_sc as plsc

_ROWS = 98304
_LANES = 128
_NW = 32
_ROWS_PER_W = _ROWS // _NW
_CHUNK = 384
_NCHUNK = _ROWS_PER_W // _CHUNK
_NBUF = 2


def _make_sc_copy():
    mesh = plsc.VectorSubcoreMesh(core_axis_name="c", subcore_axis_name="s")

    @functools.partial(
        pl.kernel,
        mesh=mesh,
        out_type=jax.ShapeDtypeStruct((_ROWS, _LANES), jnp.float32),
        scratch_types=(
            [pltpu.MemorySpace.VMEM_SHARED(
                (16, _NBUF, _CHUNK, _LANES), jnp.float32)]
            + [pltpu.SemaphoreType.DMA for _ in range(2 * _NBUF)]
        ),
    )
    def sc_copy(x_hbm, o_hbm, *scratch):
        shared = scratch[0]
        rsems = scratch[1:1 + _NBUF]
        wsems = scratch[1 + _NBUF:]
        sid = lax.axis_index("s")
        wid = lax.axis_index("c") * 16 + sid
        base = wid * _ROWS_PER_W

        def rd(g):
            return pltpu.make_async_copy(
                x_hbm.at[pl.ds(base + g * _CHUNK, _CHUNK)],
                shared.at[sid, g % _NBUF], rsems[g % _NBUF])

        def wr(g):
            return pltpu.make_async_copy(
                shared.at[sid, g % _NBUF],
                o_hbm.at[pl.ds(base + g * _CHUNK, _CHUNK)],
                wsems[g % _NBUF])

        rd(0).start()
        for g in range(_NCHUNK):
            rd(g).wait()
            wr(g).start()
            nxt = g + 1
            if nxt < _NCHUNK:
                if nxt >= _NBUF:
                    wr(nxt - _NBUF).wait()
                rd(nxt).start()
        for g in range(_NCHUNK - _NBUF, _NCHUNK):
            wr(g).wait()

    return sc_copy


_sc_copy = _make_sc_copy()


def kernel(items):
    n = items.shape[0]
    chunks = n // 128
    flat = (jnp.transpose(items, (1, 3, 0, 2))
            .reshape(2, 3, chunks, 128, 2)
            .transpose(0, 1, 2, 4, 3)
            .reshape(_ROWS, _LANES))
    out = _sc_copy(flat)
    return (out.reshape(2, 3, chunks, 2, 128)
            .transpose(0, 1, 2, 4, 3)
            .reshape(2, 3, n, 2)
            .transpose(2, 0, 3, 1))


# speedup vs baseline: 1.0510x; 1.0001x over previous
import functools

import jax
import jax.numpy as jnp
from jax import lax
from jax.experimental import pallas as pl
from jax.experimental.pallas import tpu as pltpu
from jax.experimental.pallas import tpu_sc as plsc

_ROWS = 98304
_LANES = 128
_NW = 32
_ROWS_PER_W = _ROWS // _NW
_CHUNK = 384
_NCHUNK = _ROWS_PER_W // _CHUNK
_NBUF = 2


def _make_sc_copy():
    mesh = plsc.VectorSubcoreMesh(core_axis_name="c", subcore_axis_name="s")

    @functools.partial(
        pl.kernel,
        mesh=mesh,
        out_type=jax.ShapeDtypeStruct((_ROWS, _LANES), jnp.float32),
        scratch_types=(
            [pltpu.MemorySpace.VMEM_SHARED(
                (16, _NBUF, _CHUNK, _LANES), jnp.float32)]
            + [pltpu.SemaphoreType.DMA for _ in range(2 * _NBUF)]
        ),
    )
    def sc_copy(x_hbm, o_hbm, *scratch):
        shared = scratch[0]
        rsems = scratch[1:1 + _NBUF]
        wsems = scratch[1 + _NBUF:]
        sid = lax.axis_index("s")
        core_base = lax.axis_index("c") * (_ROWS // 2)
        sub_base = core_base + sid * _CHUNK

        def rd(g):
            return pltpu.make_async_copy(
                x_hbm.at[pl.ds(sub_base + g * 16 * _CHUNK, _CHUNK)],
                shared.at[sid, g % _NBUF], rsems[g % _NBUF])

        def wr(g):
            return pltpu.make_async_copy(
                shared.at[sid, g % _NBUF],
                o_hbm.at[pl.ds(sub_base + g * 16 * _CHUNK, _CHUNK)],
                wsems[g % _NBUF])

        rd(0).start()
        for g in range(_NCHUNK):
            rd(g).wait()
            wr(g).start()
            nxt = g + 1
            if nxt < _NCHUNK:
                if nxt >= _NBUF:
                    wr(nxt - _NBUF).wait()
                rd(nxt).start()
        for g in range(_NCHUNK - _NBUF, _NCHUNK):
            wr(g).wait()

    return sc_copy


_sc_copy = _make_sc_copy()


def kernel(items):
    n = items.shape[0]
    chunks = n // 128
    flat = (jnp.transpose(items, (1, 3, 0, 2))
            .reshape(2, 3, chunks, 128, 2)
            .transpose(0, 1, 2, 4, 3)
            .reshape(_ROWS, _LANES))
    out = _sc_copy(flat)
    return (out.reshape(2, 3, chunks, 2, 128)
            .transpose(0, 1, 2, 4, 3)
            .reshape(2, 3, n, 2)
            .transpose(2, 0, 3, 1))
